# baseline (speedup 1.0000x reference)
import functools

import jax
import jax.numpy as jnp
from jax import lax
from jax.experimental import pallas as pl
from jax.experimental.pallas import tpu as pltpu
from jax.experimental.pallas import tpu_sc as plsc

N = 10000
E = 320000
IN_DIM = 128
HID1 = 256
HID2 = 256
EPS = 1e-5

NC = 2
NT = 16
CH = 64
SB = 16
NBUF = 4
EPAD = NC * NT * 10 * SB * CH
RPT = N // NT

BLK = 1000
NBLK = N // BLK

import numpy as np
_PERM32 = np.concatenate([np.arange(0, 32, 2), np.arange(1, 32, 2)])
_PERM128 = np.concatenate([32 * q + _PERM32 for q in range(4)])
_PERM256 = np.concatenate([_PERM128, 128 + _PERM128])


def _make_sc_agg(nsb, nq):
    mesh = plsc.VectorSubcoreMesh(core_axis_name="c", subcore_axis_name="s")
    npass = nq // NC

    @functools.partial(
        pl.kernel,
        mesh=mesh,
        out_type=jax.ShapeDtypeStruct((nq, N, 64), jnp.float32),
        scratch_types=[
            pltpu.VMEM((SB, CH), jnp.int32),
            pltpu.VMEM((SB, CH), jnp.int32),
            pltpu.VMEM((SB, CH), jnp.float32),
            *[pltpu.VMEM((CH, 32), jnp.int32) for _ in range(NBUF)],
            *[pltpu.VMEM((CH, 64), jnp.float32) for _ in range(2)],
            pltpu.VMEM_SHARED((N, 64), jnp.float32),
            pltpu.VMEM_SHARED((N, 32), jnp.int32),
            *[pltpu.SemaphoreType.DMA for _ in range(NBUF + 2)],
        ],
        compiler_params=pltpu.CompilerParams(
            use_tc_tiling_on_sc=False, needs_layout_passes=False),
    )
    def agg_kernel(xq, srch, dsth, probh, zrows, out,
                   srcb, dstb, probb, *bufs_table_sems):
        rowsl = bufs_table_sems[:NBUF]
        scaled = bufs_table_sems[NBUF:NBUF + 2]
        table = bufs_table_sems[NBUF + 2]
        xspm = bufs_table_sems[NBUF + 3]
        gsem = bufs_table_sems[NBUF + 4:2 * NBUF + 4]
        ssem = bufs_table_sems[2 * NBUF + 4:]

        c = lax.axis_index("c")
        s = lax.axis_index("s")
        rsl = pl.ds(s * RPT, RPT)

        def scale(j, probb, rows, outb):
            @plsc.parallel_loop(0, CH // 16, unroll=2)
            def scale_group(g):
                pvec = probb[j, pl.ds(g * 16, 16)]
                for r in range(16):
                    p = pvec[r]
                    i = g * 16 + r
                    for q in range(2):
                        w = rows[i, pl.ds(q * 16, 16)]
                        lo = plsc.bitcast(w << 16, jnp.float32)
                        hi = plsc.bitcast(w & jnp.int32(-65536), jnp.float32)
                        outb[i, pl.ds(q * 32, 16)] = lo * p
                        outb[i, pl.ds(q * 32 + 16, 16)] = hi * p

        for t in range(npass):
            qi = npass * c + t
            pltpu.sync_copy(zrows, table.at[rsl])
            pltpu.sync_copy(xq.at[qi, rsl], xspm.at[rsl])
            plsc.subcore_barrier()

            def sb_body(sb, carry):
                pltpu.sync_copy(srch.at[s, pl.ds(sb * SB, SB)], srcb)
                pltpu.sync_copy(dsth.at[s, pl.ds(sb * SB, SB)], dstb)
                pltpu.sync_copy(probh.at[s, pl.ds(sb * SB, SB)], probb)

                def ring_body(p, carry1):
                    gds = [
                        pltpu.async_copy(xspm.at[srcb.at[NBUF * p + b]],
                                         rowsl[b], gsem[b])
                        for b in range(NBUF)
                    ]
                    sds = [None, None]
                    for b in range(NBUF):
                        j = NBUF * p + b
                        gds[b].wait()
                        if b >= 2:
                            sds[b % 2].wait()
                        scale(j, probb, rowsl[b], scaled[b % 2])
                        sds[b % 2] = pltpu.async_copy(
                            scaled[b % 2], table.at[dstb.at[j]],
                            ssem[b % 2], add=True)
                    sds[0].wait()
                    sds[1].wait()
                    return carry1

                lax.fori_loop(0, SB // NBUF, ring_body, 0)
                return carry

            lax.fori_loop(0, nsb, sb_body, 0)
            plsc.subcore_barrier()
            pltpu.sync_copy(table.at[rsl], out.at[qi, rsl])

    return agg_kernel


_make_sc_agg = functools.lru_cache(maxsize=None)(_make_sc_agg)

_DN = (((1,), (1,)), ((), ()))


def _dg(a, b):
    return lax.dot_general(a, b, _DN, preferred_element_type=jnp.float32)


def _mlp1_body(x_ref, a_ref, w_ref, b_ref, h_ref, s_ref, q_ref):
    w = w_ref[...]
    h = (_dg(x_ref[...], w[:, :IN_DIM])
         + _dg(a_ref[0], w[:, IN_DIM:IN_DIM + 64])
         + _dg(a_ref[1], w[:, IN_DIM + 64:]))
    h = jnp.maximum(h + b_ref[...], 0.0)
    h_ref[...] = h

    @pl.when(pl.program_id(0) == 0)
    def _():
        s_ref[...] = jnp.zeros_like(s_ref)
        q_ref[...] = jnp.zeros_like(q_ref)

    s_ref[...] += jnp.sum(h, axis=0, keepdims=True)
    q_ref[...] += jnp.sum(h * h, axis=0, keepdims=True)


def _mlp2_body(x1_ref, a_ref, w_ref, b_ref, h_ref, s_ref, q_ref):
    w = w_ref[...]
    h = b_ref[...]
    for k in range(4):
        h = h + _dg(x1_ref[k], w[:, 64 * k:64 * (k + 1)])
        h = h + _dg(a_ref[k], w[:, 256 + 64 * k:256 + 64 * (k + 1)])
    h = jnp.maximum(h, 0.0)
    h_ref[...] = h

    @pl.when(pl.program_id(0) == 0)
    def _():
        s_ref[...] = jnp.zeros_like(s_ref)
        q_ref[...] = jnp.zeros_like(q_ref)

    s_ref[...] += jnp.sum(h, axis=0, keepdims=True)
    q_ref[...] += jnp.sum(h * h, axis=0, keepdims=True)


def _bn_split_body(h_ref, s_ref, q_ref, g_ref, be_ref, o_ref, ob_ref):
    mean = s_ref[...] / N
    var = q_ref[...] / N - mean * mean
    xn = (h_ref[...] - mean) * (lax.rsqrt(var + EPS) * g_ref[...]) + be_ref[...]
    for k in range(4):
        o_ref[k, :, :] = xn[:, 64 * k:64 * (k + 1)]
        ob_ref[k, :, :] = xn[:, 64 * k:64 * (k + 1)].astype(jnp.bfloat16)


def _bn2_out_body(h2_ref, s_ref, q_ref, g_ref, be_ref, x_ref, x1_ref,
                  wo_ref, bo_ref, out_ref):
    mean = s_ref[...] / N
    var = q_ref[...] / N - mean * mean
    x2 = (h2_ref[...] - mean) * (lax.rsqrt(var + EPS) * g_ref[...]) + be_ref[...]
    wo = wo_ref[...]
    o = _dg(x_ref[...], wo[:, 0:128]) + _dg(x2, wo[:, 384:640])
    for k in range(4):
        o = o + _dg(x1_ref[k], wo[:, 128 + 64 * k:128 + 64 * (k + 1)])
    out_ref[...] = jax.nn.sigmoid(o + bo_ref[...])


def _row_spec(d):
    return pl.BlockSpec((BLK, d), lambda i: (i, 0))


def _pair_spec(d):
    return pl.BlockSpec((2, BLK, d), lambda i: (0, i, 0))


def _quad_spec(d):
    return pl.BlockSpec((4, BLK, d), lambda i: (0, i, 0))


def _full_spec(r, d):
    return pl.BlockSpec((r, d), lambda i: (0, 0))


def kernel(x, edge_index, edge_probs, W1, b1, W2, b2, Wout, bout,
           gamma1, beta1, gamma2, beta2):
    pad = EPAD - E
    srcf = jnp.pad(edge_index[0], (0, pad))
    dstf = jnp.pad(edge_index[1], (0, pad))
    probf = jnp.pad(edge_probs, (0, pad))
    z64 = jnp.zeros((RPT, 64), jnp.float32)

    b1r = b1.reshape(1, HID1)
    b2r = b2.reshape(1, HID2)
    g1r = gamma1.reshape(1, HID1)
    be1r = beta1.reshape(1, HID1)
    g2r = gamma2.reshape(1, HID2)
    be2r = beta2.reshape(1, HID2)
    bor = bout.reshape(1, 1)

    src2 = srcf.reshape(NT, 20 * SB, CH)
    dst2 = dstf.reshape(NT, 20 * SB, CH)
    prob2 = probf.reshape(NT, 20 * SB, CH)

    xh = lax.bitcast_convert_type(
        x.astype(jnp.bfloat16).reshape(N, 2, 32, 2).transpose(1, 0, 2, 3),
        jnp.int32)
    W1p = jnp.concatenate([W1[:, :IN_DIM], W1[:, IN_DIM + _PERM128]], axis=1)
    W2p = jnp.concatenate([W2[:, :256], W2[:, 256 + _PERM256]], axis=1)

    agg1 = _make_sc_agg(20, 2)(xh, src2, dst2, prob2, z64)

    h1, s1, q1 = pl.pallas_call(
        _mlp1_body,
        grid=(NBLK,),
        in_specs=[
            _row_spec(IN_DIM), _pair_spec(64),
            _full_spec(HID1, 2 * IN_DIM), _full_spec(1, HID1),
        ],
        out_specs=[_row_spec(HID1), _full_spec(1, HID1), _full_spec(1, HID1)],
        out_shape=[
            jax.ShapeDtypeStruct((N, HID1), jnp.float32),
            jax.ShapeDtypeStruct((1, HID1), jnp.float32),
            jax.ShapeDtypeStruct((1, HID1), jnp.float32),
        ],
    )(x, agg1, W1p, b1r)

    x1s = pl.pallas_call(
        _bn_split_body,
        grid=(NBLK,),
        in_specs=[
            _row_spec(HID1), _full_spec(1, HID1), _full_spec(1, HID1),
            _full_spec(1, HID1), _full_spec(1, HID1),
        ],
        out_specs=[_quad_spec(HID1 // 4), _quad_spec(HID1 // 4)],
        out_shape=[
            jax.ShapeDtypeStruct((4, N, HID1 // 4), jnp.float32),
            jax.ShapeDtypeStruct((4, N, HID1 // 4), jnp.bfloat16),
        ],
    )(h1, s1, q1, g1r, be1r)
    x1s, x1bf = x1s

    agg2 = _make_sc_agg(20, 4)(
        lax.bitcast_convert_type(x1bf.reshape(4, N, 32, 2), jnp.int32),
        src2, dst2, prob2, z64)

    h2, s2, q2 = pl.pallas_call(
        _mlp2_body,
        grid=(NBLK,),
        in_specs=[
            _quad_spec(64), _quad_spec(64),
            _full_spec(HID2, 2 * HID1), _full_spec(1, HID2),
        ],
        out_specs=[_row_spec(HID2), _full_spec(1, HID2), _full_spec(1, HID2)],
        out_shape=[
            jax.ShapeDtypeStruct((N, HID2), jnp.float32),
            jax.ShapeDtypeStruct((1, HID2), jnp.float32),
            jax.ShapeDtypeStruct((1, HID2), jnp.float32),
        ],
    )(x1s, agg2, W2p, b2r)

    out = pl.pallas_call(
        _bn2_out_body,
        grid=(NBLK,),
        in_specs=[
            _row_spec(HID2), _full_spec(1, HID2), _full_spec(1, HID2),
            _full_spec(1, HID2), _full_spec(1, HID2),
            _row_spec(IN_DIM), _quad_spec(HID1 // 4),
            _full_spec(1, IN_DIM + HID1 + HID2), _full_spec(1, 1),
        ],
        out_specs=pl.BlockSpec((BLK, 1), lambda i: (i, 0)),
        out_shape=jax.ShapeDtypeStruct((N, 1), jnp.float32),
    )(h2, s2, q2, g2r, be2r, x, x1s, Wout, bor)

    return out


# speedup vs baseline: 1.2139x; 1.2139x over previous
import functools

import jax
import jax.numpy as jnp
from jax import lax
from jax.experimental import pallas as pl
from jax.experimental.pallas import tpu as pltpu
from jax.experimental.pallas import tpu_sc as plsc

N = 10000
E = 320000
IN_DIM = 128
HID1 = 256
HID2 = 256
EPS = 1e-5

NC = 2
NT = 16
CH = 64
SB = 16
NBUF = 8
EPAD = NC * NT * 10 * SB * CH
RPT = N // NT

BLK = 1000
NBLK = N // BLK


def _make_sc_agg(nsb, nq):
    mesh = plsc.VectorSubcoreMesh(core_axis_name="c", subcore_axis_name="s")
    npass = nq // NC

    @functools.partial(
        pl.kernel,
        mesh=mesh,
        out_type=jax.ShapeDtypeStruct((nq, N, 64), jnp.float32),
        scratch_types=[
            pltpu.VMEM((SB, CH), jnp.int32),
            pltpu.VMEM((SB, CH), jnp.int32),
            pltpu.VMEM((SB, CH), jnp.float32),
            *[pltpu.VMEM((CH, 64), jnp.float32) for _ in range(NBUF)],
            pltpu.VMEM_SHARED((N, 64), jnp.float32),
            pltpu.VMEM_SHARED((N, 64), jnp.float32),
            *[pltpu.SemaphoreType.DMA for _ in range(2 * NBUF)],
        ],
        compiler_params=pltpu.CompilerParams(
            use_tc_tiling_on_sc=False, needs_layout_passes=False),
    )
    def agg_kernel(xq, srch, dsth, probh, zrows, out,
                   srcb, dstb, probb, *bufs_table_sems):
        rowsl = bufs_table_sems[:NBUF]
        table = bufs_table_sems[NBUF]
        xspm = bufs_table_sems[NBUF + 1]
        gsem = bufs_table_sems[NBUF + 2:2 * NBUF + 2]
        ssem = bufs_table_sems[2 * NBUF + 2:]

        c = lax.axis_index("c")
        s = lax.axis_index("s")
        rsl = pl.ds(s * RPT, RPT)

        def scale(j, probb, rows):
            @plsc.parallel_loop(0, CH // 16, unroll=2)
            def scale_group(g):
                pvec = probb[j, pl.ds(g * 16, 16)]
                for r in range(16):
                    p = pvec[r]
                    i = g * 16 + r
                    for q in range(64 // 16):
                        sl = pl.ds(q * 16, 16)
                        rows[i, sl] = rows[i, sl] * p

        for t in range(npass):
            qi = npass * c + t
            pltpu.sync_copy(zrows, table.at[rsl])
            pltpu.sync_copy(xq.at[qi, rsl], xspm.at[rsl])
            plsc.subcore_barrier()

            def sb_body(sb, carry):
                pltpu.sync_copy(srch.at[s, pl.ds(sb * SB, SB)], srcb)
                pltpu.sync_copy(dsth.at[s, pl.ds(sb * SB, SB)], dstb)
                pltpu.sync_copy(probh.at[s, pl.ds(sb * SB, SB)], probb)

                def ring_body(p, carry1):
                    gds = [
                        pltpu.async_copy(xspm.at[srcb.at[NBUF * p + b]],
                                         rowsl[b], gsem[b])
                        for b in range(NBUF)
                    ]
                    sds = []
                    for b in range(NBUF):
                        j = NBUF * p + b
                        gds[b].wait()
                        scale(j, probb, rowsl[b])
                        sds.append(
                            pltpu.async_copy(rowsl[b], table.at[dstb.at[j]],
                                             ssem[b], add=True))
                    for sd in sds:
                        sd.wait()
                    return carry1

                lax.fori_loop(0, SB // NBUF, ring_body, 0)
                return carry

            lax.fori_loop(0, nsb, sb_body, 0)
            plsc.subcore_barrier()
            pltpu.sync_copy(table.at[rsl], out.at[qi, rsl])

    return agg_kernel


_make_sc_agg = functools.lru_cache(maxsize=None)(_make_sc_agg)

_DN = (((1,), (1,)), ((), ()))


def _dg(a, b):
    return lax.dot_general(a, b, _DN, preferred_element_type=jnp.float32)


def _mlp1_body(x_ref, a_ref, w_ref, b_ref, h_ref, s_ref, q_ref):
    w = w_ref[...]
    h = (_dg(x_ref[...], w[:, :IN_DIM])
         + _dg(a_ref[0], w[:, IN_DIM:IN_DIM + 64])
         + _dg(a_ref[1], w[:, IN_DIM + 64:]))
    h = jnp.maximum(h + b_ref[...], 0.0)
    h_ref[...] = h

    @pl.when(pl.program_id(0) == 0)
    def _():
        s_ref[...] = jnp.zeros_like(s_ref)
        q_ref[...] = jnp.zeros_like(q_ref)

    s_ref[...] += jnp.sum(h, axis=0, keepdims=True)
    q_ref[...] += jnp.sum(h * h, axis=0, keepdims=True)


def _mlp2_body(x1_ref, a_ref, w_ref, b_ref, h_ref, s_ref, q_ref):
    w = w_ref[...]
    h = b_ref[...]
    for k in range(4):
        h = h + _dg(x1_ref[k], w[:, 64 * k:64 * (k + 1)])
        h = h + _dg(a_ref[k], w[:, 256 + 64 * k:256 + 64 * (k + 1)])
    h = jnp.maximum(h, 0.0)
    h_ref[...] = h

    @pl.when(pl.program_id(0) == 0)
    def _():
        s_ref[...] = jnp.zeros_like(s_ref)
        q_ref[...] = jnp.zeros_like(q_ref)

    s_ref[...] += jnp.sum(h, axis=0, keepdims=True)
    q_ref[...] += jnp.sum(h * h, axis=0, keepdims=True)


def _bn_split_body(h_ref, s_ref, q_ref, g_ref, be_ref, o_ref):
    mean = s_ref[...] / N
    var = q_ref[...] / N - mean * mean
    xn = (h_ref[...] - mean) * (lax.rsqrt(var + EPS) * g_ref[...]) + be_ref[...]
    for k in range(4):
        o_ref[k, :, :] = xn[:, 64 * k:64 * (k + 1)]


def _bn2_out_body(h2_ref, s_ref, q_ref, g_ref, be_ref, x_ref, x1_ref,
                  wo_ref, bo_ref, out_ref):
    mean = s_ref[...] / N
    var = q_ref[...] / N - mean * mean
    x2 = (h2_ref[...] - mean) * (lax.rsqrt(var + EPS) * g_ref[...]) + be_ref[...]
    wo = wo_ref[...]
    o = _dg(x_ref[...], wo[:, 0:128]) + _dg(x2, wo[:, 384:640])
    for k in range(4):
        o = o + _dg(x1_ref[k], wo[:, 128 + 64 * k:128 + 64 * (k + 1)])
    out_ref[...] = jax.nn.sigmoid(o + bo_ref[...])


def _row_spec(d):
    return pl.BlockSpec((BLK, d), lambda i: (i, 0))


def _pair_spec(d):
    return pl.BlockSpec((2, BLK, d), lambda i: (0, i, 0))


def _quad_spec(d):
    return pl.BlockSpec((4, BLK, d), lambda i: (0, i, 0))


def _full_spec(r, d):
    return pl.BlockSpec((r, d), lambda i: (0, 0))


def kernel(x, edge_index, edge_probs, W1, b1, W2, b2, Wout, bout,
           gamma1, beta1, gamma2, beta2):
    pad = EPAD - E
    srcf = jnp.pad(edge_index[0], (0, pad))
    dstf = jnp.pad(edge_index[1], (0, pad))
    probf = jnp.pad(edge_probs, (0, pad))
    z64 = jnp.zeros((RPT, 64), jnp.float32)

    b1r = b1.reshape(1, HID1)
    b2r = b2.reshape(1, HID2)
    g1r = gamma1.reshape(1, HID1)
    be1r = beta1.reshape(1, HID1)
    g2r = gamma2.reshape(1, HID2)
    be2r = beta2.reshape(1, HID2)
    bor = bout.reshape(1, 1)

    src2 = srcf.reshape(NT, 20 * SB, CH)
    dst2 = dstf.reshape(NT, 20 * SB, CH)
    prob2 = probf.reshape(NT, 20 * SB, CH)

    xh = x.reshape(N, 2, 64).transpose(1, 0, 2)
    agg1 = _make_sc_agg(20, 2)(xh, src2, dst2, prob2, z64)

    h1, s1, q1 = pl.pallas_call(
        _mlp1_body,
        grid=(NBLK,),
        in_specs=[
            _row_spec(IN_DIM), _pair_spec(64),
            _full_spec(HID1, 2 * IN_DIM), _full_spec(1, HID1),
        ],
        out_specs=[_row_spec(HID1), _full_spec(1, HID1), _full_spec(1, HID1)],
        out_shape=[
            jax.ShapeDtypeStruct((N, HID1), jnp.float32),
            jax.ShapeDtypeStruct((1, HID1), jnp.float32),
            jax.ShapeDtypeStruct((1, HID1), jnp.float32),
        ],
    )(x, agg1, W1, b1r)

    x1s = pl.pallas_call(
        _bn_split_body,
        grid=(NBLK,),
        in_specs=[
            _row_spec(HID1), _full_spec(1, HID1), _full_spec(1, HID1),
            _full_spec(1, HID1), _full_spec(1, HID1),
        ],
        out_specs=_quad_spec(HID1 // 4),
        out_shape=jax.ShapeDtypeStruct((4, N, HID1 // 4), jnp.float32),
    )(h1, s1, q1, g1r, be1r)

    agg2 = _make_sc_agg(20, 4)(x1s, src2, dst2, prob2, z64)

    h2, s2, q2 = pl.pallas_call(
        _mlp2_body,
        grid=(NBLK,),
        in_specs=[
            _quad_spec(64), _quad_spec(64),
            _full_spec(HID2, 2 * HID1), _full_spec(1, HID2),
        ],
        out_specs=[_row_spec(HID2), _full_spec(1, HID2), _full_spec(1, HID2)],
        out_shape=[
            jax.ShapeDtypeStruct((N, HID2), jnp.float32),
            jax.ShapeDtypeStruct((1, HID2), jnp.float32),
            jax.ShapeDtypeStruct((1, HID2), jnp.float32),
        ],
    )(x1s, agg2, W2, b2r)

    out = pl.pallas_call(
        _bn2_out_body,
        grid=(NBLK,),
        in_specs=[
            _row_spec(HID2), _full_spec(1, HID2), _full_spec(1, HID2),
            _full_spec(1, HID2), _full_spec(1, HID2),
            _row_spec(IN_DIM), _quad_spec(HID1 // 4),
            _full_spec(1, IN_DIM + HID1 + HID2), _full_spec(1, 1),
        ],
        out_specs=pl.BlockSpec((BLK, 1), lambda i: (i, 0)),
        out_shape=jax.ShapeDtypeStruct((N, 1), jnp.float32),
    )(h2, s2, q2, g2r, be2r, x, x1s, Wout, bor)

    return out
